# Initial kernel scaffold; baseline (speedup 1.0000x reference)
import jax
import jax.numpy as jnp
from jax import lax
from jax.experimental import pallas as pl
from jax.experimental.pallas import tpu as pltpu
from jax.experimental.pallas import tpu_sc as plsc

_B = 1024
_L = 200
_WDIM = 64
_PDIM = 16
_XDIM = _WDIM + 2 * _PDIM
_N = _B * _L
_NC = 2
_NS = 16
_NW = _NC * _NS
_PER_W = _N // _NW
_C = 128
_NCHUNK = _PER_W // _C
_HT_PER_W = _B // _NW


def _sc_body(word_hbm, pos1_hbm, pos2_hbm, m0_hbm, head_hbm, tail_hbm,
             wtab_hbm, p1tab_hbm, p2tab_hbm,
             x_hbm, head_out_hbm, tail_out_hbm,
             widx_v, pidx1_v, pidx2_v, m0_v, wbuf, p1buf, p2buf,
             hidx_v, hbuf, sem):
    wid = lax.axis_index("s") * _NC + lax.axis_index("c")

    hbase = wid * _HT_PER_W
    pltpu.sync_copy(head_hbm.at[pl.ds(hbase, _HT_PER_W)], hidx_v)
    pltpu.async_copy(wtab_hbm.at[hidx_v], hbuf, sem).wait()
    pltpu.sync_copy(hbuf, head_out_hbm.at[pl.ds(hbase, _HT_PER_W)])
    pltpu.sync_copy(tail_hbm.at[pl.ds(hbase, _HT_PER_W)], hidx_v)
    pltpu.async_copy(wtab_hbm.at[hidx_v], hbuf, sem).wait()
    pltpu.sync_copy(hbuf, tail_out_hbm.at[pl.ds(hbase, _HT_PER_W)])

    base = wid * _PER_W

    def chunk(i, carry):
        off = base + i * _C
        pltpu.sync_copy(word_hbm.at[pl.ds(off, _C)], widx_v)
        pltpu.sync_copy(pos1_hbm.at[pl.ds(off, _C)], pidx1_v)
        pltpu.sync_copy(pos2_hbm.at[pl.ds(off, _C)], pidx2_v)
        pltpu.sync_copy(m0_hbm.at[pl.ds(off, _C)], m0_v)

        def mul(j, c2):
            s = pl.ds(j * 16, 16)
            m = m0_v[s]
            pidx1_v[s] = pidx1_v[s] * m
            pidx2_v[s] = pidx2_v[s] * m
            return c2

        lax.fori_loop(0, _C // 16, mul, 0)

        cw = pltpu.async_copy(wtab_hbm.at[widx_v], wbuf, sem)
        c1 = pltpu.async_copy(p1tab_hbm.at[pidx1_v], p1buf, sem)
        c2 = pltpu.async_copy(p2tab_hbm.at[pidx2_v], p2buf, sem)
        cw.wait()
        c1.wait()
        c2.wait()

        pltpu.sync_copy(wbuf, x_hbm.at[pl.ds(off, _C), pl.ds(0, _WDIM)])
        pltpu.sync_copy(p1buf, x_hbm.at[pl.ds(off, _C), pl.ds(_WDIM, _PDIM)])
        pltpu.sync_copy(p2buf, x_hbm.at[pl.ds(off, _C),
                                        pl.ds(_WDIM + _PDIM, _PDIM)])
        return carry

    lax.fori_loop(0, _NCHUNK, chunk, 0)


def kernel(word, pos1, pos2, mask, mask0, head, tail,
           word_table, pos1_table, pos2_table):
    del mask
    word_f = word.reshape(_N).astype(jnp.int32)
    pos1_f = pos1.reshape(_N).astype(jnp.int32)
    pos2_f = pos2.reshape(_N).astype(jnp.int32)
    m0_f = mask0.reshape(_N).astype(jnp.int32)
    head_i = head.astype(jnp.int32)
    tail_i = tail.astype(jnp.int32)

    mesh = plsc.VectorSubcoreMesh(core_axis_name="c", subcore_axis_name="s",
                                  num_cores=_NC, num_subcores=_NS)
    x_flat, head_e, tail_e = pl.kernel(
        _sc_body,
        out_type=(
            jax.ShapeDtypeStruct((_N, _XDIM), jnp.float32),
            jax.ShapeDtypeStruct((_B, _WDIM), jnp.float32),
            jax.ShapeDtypeStruct((_B, _WDIM), jnp.float32),
        ),
        mesh=mesh,
        scratch_types=[
            pltpu.VMEM((_C,), jnp.int32),
            pltpu.VMEM((_C,), jnp.int32),
            pltpu.VMEM((_C,), jnp.int32),
            pltpu.VMEM((_C,), jnp.int32),
            pltpu.VMEM((_C, _WDIM), jnp.float32),
            pltpu.VMEM((_C, _PDIM), jnp.float32),
            pltpu.VMEM((_C, _PDIM), jnp.float32),
            pltpu.VMEM((_HT_PER_W,), jnp.int32),
            pltpu.VMEM((_HT_PER_W, _WDIM), jnp.float32),
            pltpu.SemaphoreType.DMA,
        ],
    )(word_f, pos1_f, pos2_f, m0_f, head_i, tail_i,
      word_table, pos1_table, pos2_table)
    return x_flat.reshape(_B, _L, _XDIM), head_e, tail_e


# speedup vs baseline: 1.2913x; 1.2913x over previous
import jax
import jax.numpy as jnp
from jax import lax
from jax.experimental import pallas as pl
from jax.experimental.pallas import tpu as pltpu
from jax.experimental.pallas import tpu_sc as plsc

_B = 1024
_L = 200
_WDIM = 64
_PDIM = 16
_XDIM = _WDIM + 2 * _PDIM
_N = _B * _L
_NC = 2
_NS = 16
_NW = _NC * _NS
_PER_W = _N // _NW
_C = 128
_NCHUNK = _PER_W // _C
_HT_PER_W = _B // _NW


def _sc_body(word_hbm, pos1_hbm, pos2_hbm, m0_hbm, head_hbm, tail_hbm,
             wtab_hbm, p1tab_hbm, p2tab_hbm,
             x_hbm, head_out_hbm, tail_out_hbm,
             widx_v, pidx1_v, pidx2_v, m0_v, wbuf, p1buf, p2buf,
             hidx_v, hbuf, sem):
    wid = lax.axis_index("s") * _NC + lax.axis_index("c")

    hbase = wid * _HT_PER_W
    pltpu.sync_copy(head_hbm.at[pl.ds(hbase, _HT_PER_W)], hidx_v)
    pltpu.async_copy(wtab_hbm.at[hidx_v], hbuf, sem).wait()
    pltpu.sync_copy(hbuf, head_out_hbm.at[pl.ds(hbase, _HT_PER_W)])
    pltpu.sync_copy(tail_hbm.at[pl.ds(hbase, _HT_PER_W)], hidx_v)
    pltpu.async_copy(wtab_hbm.at[hidx_v], hbuf, sem).wait()
    pltpu.sync_copy(hbuf, tail_out_hbm.at[pl.ds(hbase, _HT_PER_W)])

    base = wid * _PER_W

    def chunk(i, carry):
        off = base + i * _C
        pltpu.sync_copy(word_hbm.at[pl.ds(off, _C)], widx_v)
        pltpu.sync_copy(pos1_hbm.at[pl.ds(off, _C)], pidx1_v)
        pltpu.sync_copy(pos2_hbm.at[pl.ds(off, _C)], pidx2_v)
        pltpu.sync_copy(m0_hbm.at[pl.ds(off, _C)], m0_v)

        def mul(j, c2):
            s = pl.ds(j * 16, 16)
            m = m0_v[s]
            pidx1_v[s] = pidx1_v[s] * m
            pidx2_v[s] = pidx2_v[s] * m
            return c2

        lax.fori_loop(0, _C // 16, mul, 0)

        cw = pltpu.async_copy(wtab_hbm.at[widx_v], wbuf, sem)
        c1 = pltpu.async_copy(p1tab_hbm.at[pidx1_v], p1buf, sem)
        c2 = pltpu.async_copy(p2tab_hbm.at[pidx2_v], p2buf, sem)
        cw.wait()
        c1.wait()
        c2.wait()

        pltpu.sync_copy(wbuf, x_hbm.at[pl.ds(off, _C), pl.ds(0, _WDIM)])
        pltpu.sync_copy(p1buf, x_hbm.at[pl.ds(off, _C), pl.ds(_WDIM, _PDIM)])
        pltpu.sync_copy(p2buf, x_hbm.at[pl.ds(off, _C),
                                        pl.ds(_WDIM + _PDIM, _PDIM)])
        return carry

    lax.fori_loop(0, _NCHUNK, chunk, 0)


def kernel(word, pos1, pos2, mask, mask0, head, tail,
           word_table, pos1_table, pos2_table):
    del mask
    word_f = word.reshape(_N).astype(jnp.int32)
    pos1_f = pos1.reshape(_N).astype(jnp.int32)
    pos2_f = pos2.reshape(_N).astype(jnp.int32)
    m0_f = mask0.reshape(_N).astype(jnp.int32)
    head_i = head.astype(jnp.int32)
    tail_i = tail.astype(jnp.int32)

    mesh = plsc.VectorSubcoreMesh(core_axis_name="c", subcore_axis_name="s",
                                  num_cores=_NC, num_subcores=_NS)
    x_flat, head_e, tail_e = pl.kernel(
        _sc_body,
        out_type=(
            jax.ShapeDtypeStruct((_N, _XDIM), jnp.float32),
            jax.ShapeDtypeStruct((_B, _WDIM), jnp.float32),
            jax.ShapeDtypeStruct((_B, _WDIM), jnp.float32),
        ),
        mesh=mesh,
        compiler_params=pltpu.CompilerParams(use_tc_tiling_on_sc=False),
        scratch_types=[
            pltpu.VMEM((_C,), jnp.int32),
            pltpu.VMEM((_C,), jnp.int32),
            pltpu.VMEM((_C,), jnp.int32),
            pltpu.VMEM((_C,), jnp.int32),
            pltpu.VMEM((_C, _WDIM), jnp.float32),
            pltpu.VMEM((_C, _PDIM), jnp.float32),
            pltpu.VMEM((_C, _PDIM), jnp.float32),
            pltpu.VMEM((_HT_PER_W,), jnp.int32),
            pltpu.VMEM((_HT_PER_W, _WDIM), jnp.float32),
            pltpu.SemaphoreType.DMA,
        ],
    )(word_f, pos1_f, pos2_f, m0_f, head_i, tail_i,
      word_table, pos1_table, pos2_table)
    return x_flat.reshape(_B, _L, _XDIM), head_e, tail_e
